# baseline (speedup 1.0000x reference)
import jax
import jax.numpy as jnp
from jax.experimental import pallas as pl

_ROWS = 32
_COLS = 1_000_000
_NOISE_SCALE = 0.1
_CHUNK = 65536
_NCHUNK = -(-_COLS // _CHUNK)
_K = 16
_EPS = 0.02
_IMAX = jnp.iinfo(jnp.int32).max

_consts_cache = None


def _consts():
    global _consts_cache
    if _consts_cache is None:
        nkey = jax.random.fold_in(jax.random.key(0), 1)
        noise = _NOISE_SCALE * jax.random.normal(
            nkey, (_ROWS, _COLS), dtype=jnp.float32)
        tv, ti = jax.lax.top_k(noise, _K + 1)
        delta = tv[:, 0:1] - tv[:, _K:_K + 1]
        lnthr = jnp.log(jnp.maximum(delta, 1e-30)) - _EPS
        _consts_cache = {
            "noise": noise,
            "lnthr": jnp.broadcast_to(lnthr, (_ROWS, 128)),
            "topn": tv[:, :_K],
            "topi": ti[:, :_K].astype(jnp.int32),
        }
    return _consts_cache


def _stats_kernel(x_ref, m_ref, s_ref):
    c = pl.program_id(0)
    x = x_ref[...]
    col = jax.lax.broadcasted_iota(jnp.int32, x.shape, 1) + c * _CHUNK
    x = jnp.where(col < _COLS, x, -jnp.inf)

    @pl.when(c == 0)
    def _():
        m_ref[...] = jnp.full_like(m_ref, -jnp.inf)
        s_ref[...] = jnp.zeros_like(s_ref)

    m_old = m_ref[...]
    cmax = jnp.max(x, axis=1, keepdims=True)
    m_new = jnp.maximum(m_old, cmax)
    e = jnp.exp(x - m_new[:, :1])
    csum = jnp.sum(e, axis=1, keepdims=True)
    s_ref[...] = s_ref[...] * jnp.exp(m_old - m_new) + csum
    m_ref[...] = m_new


def _filter_kernel(m_ref, s_ref, t_ref, x_ref, cnt_ref, bx_ref, bi_ref):
    c = pl.program_id(0)
    x = x_ref[...]
    col = jax.lax.broadcasted_iota(jnp.int32, x.shape, 1) + c * _CHUNK
    x = jnp.where(col < _COLS, x, -jnp.inf)
    theta = (m_ref[...][:, :1] + jnp.log(s_ref[...][:, :1])
             + t_ref[...][:, :1])
    mask = x >= theta
    cnt = jnp.sum(mask.astype(jnp.int32), axis=1, keepdims=True)
    bx = jnp.max(jnp.where(mask, x, -jnp.inf), axis=1, keepdims=True)
    bi = jnp.min(jnp.where(mask & (x == bx), col, _IMAX),
                 axis=1, keepdims=True)
    cnt_ref[0] = jnp.broadcast_to(cnt, cnt_ref.shape[1:])
    bx_ref[0] = jnp.broadcast_to(bx, bx_ref.shape[1:])
    bi_ref[0] = jnp.broadcast_to(bi, bi_ref.shape[1:])


def _merge_kernel(m_ref, s_ref, bx_ref, bi_ref, nb_ref, xk_ref, nk_ref,
                  ik_ref, out_ref):
    m = m_ref[...][:, :1]
    inv_s = 1.0 / s_ref[...][:, :1]
    lane = jax.lax.broadcasted_iota(jnp.int32, (_ROWS, 128), 1)
    bx = bx_ref[...]
    v1 = jnp.where(bx > -jnp.inf,
                   jnp.exp(bx - m) * inv_s + nb_ref[...], -jnp.inf)
    v2 = jnp.where(lane < _K,
                   jnp.exp(xk_ref[...] - m) * inv_s + nk_ref[...],
                   -jnp.inf)
    vm = jnp.maximum(jnp.max(v1, axis=1, keepdims=True),
                     jnp.max(v2, axis=1, keepdims=True))
    i1 = jnp.min(jnp.where(v1 == vm, bi_ref[...], _IMAX),
                 axis=1, keepdims=True)
    i2 = jnp.min(jnp.where(v2 == vm, ik_ref[...], _IMAX),
                 axis=1, keepdims=True)
    out_ref[...] = jnp.broadcast_to(jnp.minimum(i1, i2), out_ref.shape)


def _argmax_kernel(m_ref, s_ref, x_ref, n_ref, bi_ref, bv_ref):
    c = pl.program_id(0)
    x = x_ref[...]
    n = n_ref[...]
    col = jax.lax.broadcasted_iota(jnp.int32, x.shape, 1) + c * _CHUNK
    m = m_ref[...][:, :1]
    inv_s = 1.0 / s_ref[...][:, :1]
    v = jnp.exp(x - m) * inv_s + n
    v = jnp.where(col < _COLS, v, -jnp.inf)
    cmax = jnp.max(v, axis=1, keepdims=True)
    idxs = jnp.where(v == cmax, col, _IMAX)
    carg = jnp.min(idxs, axis=1, keepdims=True)

    @pl.when(c == 0)
    def _():
        bv_ref[...] = jnp.full_like(bv_ref, -jnp.inf)
        bi_ref[...] = jnp.zeros_like(bi_ref)

    bv = bv_ref[...][:, :1]
    bi = bi_ref[...][:, :1]
    upd = cmax > bv
    bv_ref[...] = jnp.broadcast_to(jnp.where(upd, cmax, bv), bv_ref.shape)
    bi_ref[...] = jnp.broadcast_to(jnp.where(upd, carg, bi), bi_ref.shape)


def _small_spec():
    return pl.BlockSpec((_ROWS, 128), lambda *_: (0, 0))


def kernel(logits):
    cst = _consts()
    m, s = pl.pallas_call(
        _stats_kernel,
        grid=(_NCHUNK,),
        in_specs=[pl.BlockSpec((_ROWS, _CHUNK), lambda c: (0, c))],
        out_specs=[_small_spec(), _small_spec()],
        out_shape=[
            jax.ShapeDtypeStruct((_ROWS, 128), jnp.float32),
            jax.ShapeDtypeStruct((_ROWS, 128), jnp.float32),
        ],
    )(logits)

    cnt, bx, bi = pl.pallas_call(
        _filter_kernel,
        grid=(_NCHUNK,),
        in_specs=[
            _small_spec(), _small_spec(), _small_spec(),
            pl.BlockSpec((_ROWS, _CHUNK), lambda c: (0, c)),
        ],
        out_specs=[pl.BlockSpec((1, _ROWS, 128), lambda c: (c, 0, 0))] * 3,
        out_shape=[
            jax.ShapeDtypeStruct((_NCHUNK, _ROWS, 128), jnp.int32),
            jax.ShapeDtypeStruct((_NCHUNK, _ROWS, 128), jnp.float32),
            jax.ShapeDtypeStruct((_NCHUNK, _ROWS, 128), jnp.int32),
        ],
    )(m, s, cst["lnthr"], logits)

    cnt0 = cnt[:, :, 0]
    bx0 = bx[:, :, 0].T
    bi0 = bi[:, :, 0].T
    need_fallback = jnp.any(cnt0 >= 2)

    nb = jnp.take_along_axis(
        cst["noise"], jnp.clip(bi0, 0, _COLS - 1), axis=1)
    xk = jnp.take_along_axis(logits, cst["topi"], axis=1)

    def _pad(a, fill, dtype):
        out = jnp.full((_ROWS, 128), fill, dtype)
        return out.at[:, : a.shape[1]].set(a.astype(dtype))

    bx_p = _pad(bx0, -jnp.inf, jnp.float32)
    bi_p = _pad(bi0, _IMAX, jnp.int32)
    nb_p = _pad(nb, 0.0, jnp.float32)
    xk_p = _pad(xk, 0.0, jnp.float32)
    nk_p = _pad(cst["topn"], 0.0, jnp.float32)
    ik_p = _pad(cst["topi"], _IMAX, jnp.int32)

    idx_fast = pl.pallas_call(
        _merge_kernel,
        grid=(1,),
        in_specs=[_small_spec()] * 8,
        out_specs=_small_spec(),
        out_shape=jax.ShapeDtypeStruct((_ROWS, 128), jnp.int32),
    )(m, s, bx_p, bi_p, nb_p, xk_p, nk_p, ik_p)[:, 0]

    def _full(_):
        bi_f, _bv = pl.pallas_call(
            _argmax_kernel,
            grid=(_NCHUNK,),
            in_specs=[
                _small_spec(), _small_spec(),
                pl.BlockSpec((_ROWS, _CHUNK), lambda c: (0, c)),
                pl.BlockSpec((_ROWS, _CHUNK), lambda c: (0, c)),
            ],
            out_specs=[_small_spec(), _small_spec()],
            out_shape=[
                jax.ShapeDtypeStruct((_ROWS, 128), jnp.int32),
                jax.ShapeDtypeStruct((_ROWS, 128), jnp.float32),
            ],
        )(m, s, logits, cst["noise"])
        return bi_f[:, 0]

    return jax.lax.cond(need_fallback, _full, lambda _: idx_fast,
                        operand=None)


# speedup vs baseline: 206.6977x; 206.6977x over previous
import jax
import jax.numpy as jnp
from jax.experimental import pallas as pl

_ROWS = 32
_COLS = 1_000_000
_NOISE_SCALE = 0.1
_CHUNK = 65536
_NCHUNK = -(-_COLS // _CHUNK)
_K = 16
_EPS = 0.02
_IMAX = jnp.iinfo(jnp.int32).max

_consts_cache = None


def _consts():
    global _consts_cache
    if _consts_cache is None:
        with jax.ensure_compile_time_eval():
            nkey = jax.random.fold_in(jax.random.key(0), 1)
            noise = _NOISE_SCALE * jax.random.normal(
                nkey, (_ROWS, _COLS), dtype=jnp.float32)
            tv, ti = jax.lax.top_k(noise, _K + 1)
            delta = tv[:, 0:1] - tv[:, _K:_K + 1]
            lnthr = jnp.log(jnp.maximum(delta, 1e-30)) - _EPS
            _consts_cache = {
                "noise": noise,
                "lnthr": jnp.broadcast_to(lnthr, (_ROWS, 128)),
                "topn": tv[:, :_K],
                "topi": ti[:, :_K].astype(jnp.int32),
            }
    return _consts_cache


def _stats_kernel(x_ref, m_ref, s_ref):
    c = pl.program_id(0)
    x = x_ref[...]
    col = jax.lax.broadcasted_iota(jnp.int32, x.shape, 1) + c * _CHUNK
    x = jnp.where(col < _COLS, x, -jnp.inf)

    @pl.when(c == 0)
    def _():
        m_ref[...] = jnp.full_like(m_ref, -jnp.inf)
        s_ref[...] = jnp.zeros_like(s_ref)

    m_old = m_ref[...]
    cmax = jnp.max(x, axis=1, keepdims=True)
    m_new = jnp.maximum(m_old, cmax)
    e = jnp.exp(x - m_new[:, :1])
    csum = jnp.sum(e, axis=1, keepdims=True)
    s_ref[...] = s_ref[...] * jnp.exp(m_old - m_new) + csum
    m_ref[...] = m_new


def _filter_kernel(m_ref, s_ref, t_ref, x_ref, cnt_ref, bx_ref, bi_ref):
    c = pl.program_id(0)
    x = x_ref[...]
    col = jax.lax.broadcasted_iota(jnp.int32, x.shape, 1) + c * _CHUNK
    x = jnp.where(col < _COLS, x, -jnp.inf)
    theta = (m_ref[...][:, :1] + jnp.log(s_ref[...][:, :1])
             + t_ref[...][:, :1])
    mask = x >= theta
    cnt = jnp.sum(mask.astype(jnp.int32), axis=1, keepdims=True)
    bx = jnp.max(jnp.where(mask, x, -jnp.inf), axis=1, keepdims=True)
    bi = jnp.min(jnp.where(mask & (x == bx), col, _IMAX),
                 axis=1, keepdims=True)
    cnt_ref[0] = jnp.broadcast_to(cnt, cnt_ref.shape[1:])
    bx_ref[0] = jnp.broadcast_to(bx, bx_ref.shape[1:])
    bi_ref[0] = jnp.broadcast_to(bi, bi_ref.shape[1:])


def _merge_kernel(m_ref, s_ref, bx_ref, bi_ref, nb_ref, xk_ref, nk_ref,
                  ik_ref, out_ref):
    m = m_ref[...][:, :1]
    inv_s = 1.0 / s_ref[...][:, :1]
    lane = jax.lax.broadcasted_iota(jnp.int32, (_ROWS, 128), 1)
    bx = bx_ref[...]
    v1 = jnp.where(bx > -jnp.inf,
                   jnp.exp(bx - m) * inv_s + nb_ref[...], -jnp.inf)
    v2 = jnp.where(lane < _K,
                   jnp.exp(xk_ref[...] - m) * inv_s + nk_ref[...],
                   -jnp.inf)
    vm = jnp.maximum(jnp.max(v1, axis=1, keepdims=True),
                     jnp.max(v2, axis=1, keepdims=True))
    i1 = jnp.min(jnp.where(v1 == vm, bi_ref[...], _IMAX),
                 axis=1, keepdims=True)
    i2 = jnp.min(jnp.where(v2 == vm, ik_ref[...], _IMAX),
                 axis=1, keepdims=True)
    out_ref[...] = jnp.broadcast_to(jnp.minimum(i1, i2), out_ref.shape)


def _argmax_kernel(m_ref, s_ref, x_ref, n_ref, bi_ref, bv_ref):
    c = pl.program_id(0)
    x = x_ref[...]
    n = n_ref[...]
    col = jax.lax.broadcasted_iota(jnp.int32, x.shape, 1) + c * _CHUNK
    m = m_ref[...][:, :1]
    inv_s = 1.0 / s_ref[...][:, :1]
    v = jnp.exp(x - m) * inv_s + n
    v = jnp.where(col < _COLS, v, -jnp.inf)
    cmax = jnp.max(v, axis=1, keepdims=True)
    idxs = jnp.where(v == cmax, col, _IMAX)
    carg = jnp.min(idxs, axis=1, keepdims=True)

    @pl.when(c == 0)
    def _():
        bv_ref[...] = jnp.full_like(bv_ref, -jnp.inf)
        bi_ref[...] = jnp.zeros_like(bi_ref)

    bv = bv_ref[...][:, :1]
    bi = bi_ref[...][:, :1]
    upd = cmax > bv
    bv_ref[...] = jnp.broadcast_to(jnp.where(upd, cmax, bv), bv_ref.shape)
    bi_ref[...] = jnp.broadcast_to(jnp.where(upd, carg, bi), bi_ref.shape)


def _small_spec():
    return pl.BlockSpec((_ROWS, 128), lambda *_: (0, 0))


def kernel(logits):
    cst = _consts()
    m, s = pl.pallas_call(
        _stats_kernel,
        grid=(_NCHUNK,),
        in_specs=[pl.BlockSpec((_ROWS, _CHUNK), lambda c: (0, c))],
        out_specs=[_small_spec(), _small_spec()],
        out_shape=[
            jax.ShapeDtypeStruct((_ROWS, 128), jnp.float32),
            jax.ShapeDtypeStruct((_ROWS, 128), jnp.float32),
        ],
    )(logits)

    cnt, bx, bi = pl.pallas_call(
        _filter_kernel,
        grid=(_NCHUNK,),
        in_specs=[
            _small_spec(), _small_spec(), _small_spec(),
            pl.BlockSpec((_ROWS, _CHUNK), lambda c: (0, c)),
        ],
        out_specs=[pl.BlockSpec((1, _ROWS, 128), lambda c: (c, 0, 0))] * 3,
        out_shape=[
            jax.ShapeDtypeStruct((_NCHUNK, _ROWS, 128), jnp.int32),
            jax.ShapeDtypeStruct((_NCHUNK, _ROWS, 128), jnp.float32),
            jax.ShapeDtypeStruct((_NCHUNK, _ROWS, 128), jnp.int32),
        ],
    )(m, s, cst["lnthr"], logits)

    cnt0 = cnt[:, :, 0]
    bx0 = bx[:, :, 0].T
    bi0 = bi[:, :, 0].T
    need_fallback = jnp.any(cnt0 >= 2)

    nb = jnp.take_along_axis(
        cst["noise"], jnp.clip(bi0, 0, _COLS - 1), axis=1)
    xk = jnp.take_along_axis(logits, cst["topi"], axis=1)

    def _pad(a, fill, dtype):
        out = jnp.full((_ROWS, 128), fill, dtype)
        return out.at[:, : a.shape[1]].set(a.astype(dtype))

    bx_p = _pad(bx0, -jnp.inf, jnp.float32)
    bi_p = _pad(bi0, _IMAX, jnp.int32)
    nb_p = _pad(nb, 0.0, jnp.float32)
    xk_p = _pad(xk, 0.0, jnp.float32)
    nk_p = _pad(cst["topn"], 0.0, jnp.float32)
    ik_p = _pad(cst["topi"], _IMAX, jnp.int32)

    idx_fast = pl.pallas_call(
        _merge_kernel,
        grid=(1,),
        in_specs=[_small_spec()] * 8,
        out_specs=_small_spec(),
        out_shape=jax.ShapeDtypeStruct((_ROWS, 128), jnp.int32),
    )(m, s, bx_p, bi_p, nb_p, xk_p, nk_p, ik_p)[:, 0]

    def _full(_):
        bi_f, _bv = pl.pallas_call(
            _argmax_kernel,
            grid=(_NCHUNK,),
            in_specs=[
                _small_spec(), _small_spec(),
                pl.BlockSpec((_ROWS, _CHUNK), lambda c: (0, c)),
                pl.BlockSpec((_ROWS, _CHUNK), lambda c: (0, c)),
            ],
            out_specs=[_small_spec(), _small_spec()],
            out_shape=[
                jax.ShapeDtypeStruct((_ROWS, 128), jnp.int32),
                jax.ShapeDtypeStruct((_ROWS, 128), jnp.float32),
            ],
        )(m, s, logits, cst["noise"])
        return bi_f[:, 0]

    return jax.lax.cond(need_fallback, _full, lambda _: idx_fast, None)


# speedup vs baseline: 291.6412x; 1.4110x over previous
import jax
import jax.numpy as jnp
from jax.experimental import pallas as pl

_ROWS = 32
_COLS = 1_000_000
_NOISE_SCALE = 0.1
_CHUNK = 65536
_NCHUNK = -(-_COLS // _CHUNK)
_K = 16
_EPS = 0.02
_IMAX = jnp.iinfo(jnp.int32).max

_consts_cache = None


def _consts():
    global _consts_cache
    if _consts_cache is None:
        with jax.ensure_compile_time_eval():
            nkey = jax.random.fold_in(jax.random.key(0), 1)
            noise = _NOISE_SCALE * jax.random.normal(
                nkey, (_ROWS, _COLS), dtype=jnp.float32)
            tv, ti = jax.lax.top_k(noise, _K + 1)
            delta = tv[:, 0:1] - tv[:, _K:_K + 1]
            lnthr = jnp.log(jnp.maximum(delta, 1e-30)) - _EPS
            _consts_cache = {
                "noise": noise,
                "lnthr": jnp.broadcast_to(lnthr, (_ROWS, 128)),
                "topn": tv[:, :_K],
                "topi": ti[:, :_K].astype(jnp.int32),
            }
    return _consts_cache


def _fused_kernel(x_ref, m_ref, s_ref, cm_ref, ci_ref, c2_ref):
    c = pl.program_id(0)
    x = x_ref[...]
    col = jax.lax.broadcasted_iota(jnp.int32, x.shape, 1) + c * _CHUNK
    x = jnp.where(col < _COLS, x, -jnp.inf)

    @pl.when(c == 0)
    def _():
        m_ref[...] = jnp.full_like(m_ref, -jnp.inf)
        s_ref[...] = jnp.zeros_like(s_ref)

    cmax = jnp.max(x, axis=1, keepdims=True)
    m_old = m_ref[...]
    m_new = jnp.maximum(m_old, cmax)
    e = jnp.exp(x - m_new[:, :1])
    csum = jnp.sum(e, axis=1, keepdims=True)
    s_ref[...] = s_ref[...] * jnp.exp(m_old - m_new) + csum
    m_ref[...] = m_new
    carg = jnp.min(jnp.where(x == cmax, col, _IMAX),
                   axis=1, keepdims=True)
    cmax2 = jnp.max(jnp.where(col == carg, -jnp.inf, x),
                    axis=1, keepdims=True)
    cm_ref[0] = jnp.broadcast_to(cmax, cm_ref.shape[1:])
    ci_ref[0] = jnp.broadcast_to(carg, ci_ref.shape[1:])
    c2_ref[0] = jnp.broadcast_to(cmax2, c2_ref.shape[1:])


def _merge_kernel(m_ref, s_ref, t_ref, cm_ref, ci_ref, c2_ref, nb_ref,
                  xk_ref, nk_ref, ik_ref, out_ref, fb_ref):
    m = m_ref[...][:, :1]
    inv_s = 1.0 / s_ref[...][:, :1]
    theta = m + jnp.log(s_ref[...][:, :1]) + t_ref[...][:, :1]
    lane = jax.lax.broadcasted_iota(jnp.int32, (_ROWS, 128), 1)
    cm = cm_ref[...]
    v1 = jnp.where(cm >= theta,
                   jnp.exp(cm - m) * inv_s + nb_ref[...], -jnp.inf)
    v2 = jnp.where(lane < _K,
                   jnp.exp(xk_ref[...] - m) * inv_s + nk_ref[...],
                   -jnp.inf)
    vm = jnp.maximum(jnp.max(v1, axis=1, keepdims=True),
                     jnp.max(v2, axis=1, keepdims=True))
    i1 = jnp.min(jnp.where(v1 == vm, ci_ref[...], _IMAX),
                 axis=1, keepdims=True)
    i2 = jnp.min(jnp.where(v2 == vm, ik_ref[...], _IMAX),
                 axis=1, keepdims=True)
    out_ref[...] = jnp.broadcast_to(jnp.minimum(i1, i2), out_ref.shape)
    fb = jnp.any(c2_ref[...] >= theta)
    fb_ref[...] = jnp.broadcast_to(fb.astype(jnp.int32), fb_ref.shape)


def _argmax_kernel(m_ref, s_ref, x_ref, n_ref, bi_ref, bv_ref):
    c = pl.program_id(0)
    x = x_ref[...]
    n = n_ref[...]
    col = jax.lax.broadcasted_iota(jnp.int32, x.shape, 1) + c * _CHUNK
    m = m_ref[...][:, :1]
    inv_s = 1.0 / s_ref[...][:, :1]
    v = jnp.exp(x - m) * inv_s + n
    v = jnp.where(col < _COLS, v, -jnp.inf)
    cmax = jnp.max(v, axis=1, keepdims=True)
    idxs = jnp.where(v == cmax, col, _IMAX)
    carg = jnp.min(idxs, axis=1, keepdims=True)

    @pl.when(c == 0)
    def _():
        bv_ref[...] = jnp.full_like(bv_ref, -jnp.inf)
        bi_ref[...] = jnp.zeros_like(bi_ref)

    bv = bv_ref[...][:, :1]
    bi = bi_ref[...][:, :1]
    upd = cmax > bv
    bv_ref[...] = jnp.broadcast_to(jnp.where(upd, cmax, bv), bv_ref.shape)
    bi_ref[...] = jnp.broadcast_to(jnp.where(upd, carg, bi), bi_ref.shape)


def _small_spec():
    return pl.BlockSpec((_ROWS, 128), lambda *_: (0, 0))


def kernel(logits):
    cst = _consts()
    m, s, cm, ci, c2 = pl.pallas_call(
        _fused_kernel,
        grid=(_NCHUNK,),
        in_specs=[pl.BlockSpec((_ROWS, _CHUNK), lambda c: (0, c))],
        out_specs=[_small_spec(), _small_spec()]
        + [pl.BlockSpec((1, _ROWS, 128), lambda c: (c, 0, 0))] * 3,
        out_shape=[
            jax.ShapeDtypeStruct((_ROWS, 128), jnp.float32),
            jax.ShapeDtypeStruct((_ROWS, 128), jnp.float32),
            jax.ShapeDtypeStruct((_NCHUNK, _ROWS, 128), jnp.float32),
            jax.ShapeDtypeStruct((_NCHUNK, _ROWS, 128), jnp.int32),
            jax.ShapeDtypeStruct((_NCHUNK, _ROWS, 128), jnp.float32),
        ],
    )(logits)

    cm0 = cm[:, :, 0].T
    ci0 = ci[:, :, 0].T
    c20 = c2[:, :, 0].T

    nb = jnp.take_along_axis(cst["noise"], ci0, axis=1)
    xk = jnp.take_along_axis(logits, cst["topi"], axis=1)

    def _pad(a, fill, dtype):
        out = jnp.full((_ROWS, 128), fill, dtype)
        return out.at[:, : a.shape[1]].set(a.astype(dtype))

    cm_p = _pad(cm0, -jnp.inf, jnp.float32)
    ci_p = _pad(ci0, _IMAX, jnp.int32)
    c2_p = _pad(c20, -jnp.inf, jnp.float32)
    nb_p = _pad(nb, 0.0, jnp.float32)
    xk_p = _pad(xk, 0.0, jnp.float32)
    nk_p = _pad(cst["topn"], 0.0, jnp.float32)
    ik_p = _pad(cst["topi"], _IMAX, jnp.int32)

    idx_fast, fb = pl.pallas_call(
        _merge_kernel,
        grid=(1,),
        in_specs=[_small_spec()] * 10,
        out_specs=[_small_spec(), _small_spec()],
        out_shape=[
            jax.ShapeDtypeStruct((_ROWS, 128), jnp.int32),
            jax.ShapeDtypeStruct((_ROWS, 128), jnp.int32),
        ],
    )(m, s, cst["lnthr"], cm_p, ci_p, c2_p, nb_p, xk_p, nk_p, ik_p)
    idx_fast = idx_fast[:, 0]
    need_fallback = fb[0, 0] > 0

    def _full(_):
        bi_f, _bv = pl.pallas_call(
            _argmax_kernel,
            grid=(_NCHUNK,),
            in_specs=[
                _small_spec(), _small_spec(),
                pl.BlockSpec((_ROWS, _CHUNK), lambda c: (0, c)),
                pl.BlockSpec((_ROWS, _CHUNK), lambda c: (0, c)),
            ],
            out_specs=[_small_spec(), _small_spec()],
            out_shape=[
                jax.ShapeDtypeStruct((_ROWS, 128), jnp.int32),
                jax.ShapeDtypeStruct((_ROWS, 128), jnp.float32),
            ],
        )(m, s, logits, cst["noise"])
        return bi_f[:, 0]

    return jax.lax.cond(need_fallback, _full, lambda _: idx_fast, None)


# speedup vs baseline: 472.3146x; 1.6195x over previous
import jax
import jax.numpy as jnp
from jax.experimental import pallas as pl

_ROWS = 32
_COLS = 1_000_000
_NOISE_SCALE = 0.1
_CHUNK = 65536
_NCHUNK = -(-_COLS // _CHUNK)
_K = 16
_EPS = 0.02
_IMAX = jnp.iinfo(jnp.int32).max

_consts_cache = None


def _consts():
    global _consts_cache
    if _consts_cache is None:
        with jax.ensure_compile_time_eval():
            nkey = jax.random.fold_in(jax.random.key(0), 1)
            noise = _NOISE_SCALE * jax.random.normal(
                nkey, (_ROWS, _COLS), dtype=jnp.float32)
            tv, ti = jax.lax.top_k(noise, _K + 1)
            delta = tv[:, 0:1] - tv[:, _K:_K + 1]
            lnthr = jnp.log(jnp.maximum(delta, 1e-30)) - _EPS
            _consts_cache = {
                "noise": noise,
                "lnthr": jnp.broadcast_to(lnthr, (_ROWS, 128)),
                "topn": tv[:, :_K],
                "topi": ti[:, :_K].astype(jnp.int32),
            }
    return _consts_cache


def _stream_kernel(x_ref, m_ref, s_ref, cm_ref):
    c = pl.program_id(0)
    x = x_ref[...]
    col = jax.lax.broadcasted_iota(jnp.int32, x.shape, 1) + c * _CHUNK
    x = jnp.where(col < _COLS, x, -jnp.inf)

    @pl.when(c == 0)
    def _():
        m_ref[...] = jnp.full_like(m_ref, -jnp.inf)
        s_ref[...] = jnp.zeros_like(s_ref)

    cmax = jnp.max(x, axis=1, keepdims=True)
    m_old = m_ref[...]
    m_new = jnp.maximum(m_old, cmax)
    e = jnp.exp(x - m_new[:, :1])
    csum = jnp.sum(e, axis=1, keepdims=True)
    s_ref[...] = s_ref[...] * jnp.exp(m_old - m_new) + csum
    m_ref[...] = m_new
    cm_ref[0] = jnp.broadcast_to(cmax, cm_ref.shape[1:])


def _merge_kernel(m_ref, s_ref, t_ref, cm_ref, xk_ref, nk_ref, ik_ref,
                  out_ref, fb_ref):
    m = m_ref[...][:, :1]
    inv_s = 1.0 / s_ref[...][:, :1]
    theta = m + jnp.log(s_ref[...][:, :1]) + t_ref[...][:, :1]
    lane = jax.lax.broadcasted_iota(jnp.int32, (_ROWS, 128), 1)
    v2 = jnp.where(lane < _K,
                   jnp.exp(xk_ref[...] - m) * inv_s + nk_ref[...],
                   -jnp.inf)
    vm = jnp.max(v2, axis=1, keepdims=True)
    i2 = jnp.min(jnp.where(v2 == vm, ik_ref[...], _IMAX),
                 axis=1, keepdims=True)
    out_ref[...] = jnp.broadcast_to(i2, out_ref.shape)
    fb = jnp.any(cm_ref[...] >= theta)
    fb_ref[...] = jnp.broadcast_to(fb.astype(jnp.int32), fb_ref.shape)


def _argmax_kernel(m_ref, s_ref, x_ref, n_ref, bi_ref, bv_ref):
    c = pl.program_id(0)
    x = x_ref[...]
    n = n_ref[...]
    col = jax.lax.broadcasted_iota(jnp.int32, x.shape, 1) + c * _CHUNK
    m = m_ref[...][:, :1]
    inv_s = 1.0 / s_ref[...][:, :1]
    v = jnp.exp(x - m) * inv_s + n
    v = jnp.where(col < _COLS, v, -jnp.inf)
    cmax = jnp.max(v, axis=1, keepdims=True)
    idxs = jnp.where(v == cmax, col, _IMAX)
    carg = jnp.min(idxs, axis=1, keepdims=True)

    @pl.when(c == 0)
    def _():
        bv_ref[...] = jnp.full_like(bv_ref, -jnp.inf)
        bi_ref[...] = jnp.zeros_like(bi_ref)

    bv = bv_ref[...][:, :1]
    bi = bi_ref[...][:, :1]
    upd = cmax > bv
    bv_ref[...] = jnp.broadcast_to(jnp.where(upd, cmax, bv), bv_ref.shape)
    bi_ref[...] = jnp.broadcast_to(jnp.where(upd, carg, bi), bi_ref.shape)


def _small_spec():
    return pl.BlockSpec((_ROWS, 128), lambda *_: (0, 0))


def kernel(logits):
    cst = _consts()
    m, s, cm = pl.pallas_call(
        _stream_kernel,
        grid=(_NCHUNK,),
        in_specs=[pl.BlockSpec((_ROWS, _CHUNK), lambda c: (0, c))],
        out_specs=[_small_spec(), _small_spec(),
                   pl.BlockSpec((1, _ROWS, 128), lambda c: (c, 0, 0))],
        out_shape=[
            jax.ShapeDtypeStruct((_ROWS, 128), jnp.float32),
            jax.ShapeDtypeStruct((_ROWS, 128), jnp.float32),
            jax.ShapeDtypeStruct((_NCHUNK, _ROWS, 128), jnp.float32),
        ],
    )(logits)

    cm0 = cm[:, :, 0].T
    xk = jnp.take_along_axis(logits, cst["topi"], axis=1)

    def _pad(a, fill, dtype):
        out = jnp.full((_ROWS, 128), fill, dtype)
        return out.at[:, : a.shape[1]].set(a.astype(dtype))

    cm_p = _pad(cm0, -jnp.inf, jnp.float32)
    xk_p = _pad(xk, 0.0, jnp.float32)
    nk_p = _pad(cst["topn"], 0.0, jnp.float32)
    ik_p = _pad(cst["topi"], _IMAX, jnp.int32)

    idx_fast, fb = pl.pallas_call(
        _merge_kernel,
        grid=(1,),
        in_specs=[_small_spec()] * 7,
        out_specs=[_small_spec(), _small_spec()],
        out_shape=[
            jax.ShapeDtypeStruct((_ROWS, 128), jnp.int32),
            jax.ShapeDtypeStruct((_ROWS, 128), jnp.int32),
        ],
    )(m, s, cst["lnthr"], cm_p, xk_p, nk_p, ik_p)
    idx_fast = idx_fast[:, 0]
    need_fallback = fb[0, 0] > 0

    def _full(_):
        bi_f, _bv = pl.pallas_call(
            _argmax_kernel,
            grid=(_NCHUNK,),
            in_specs=[
                _small_spec(), _small_spec(),
                pl.BlockSpec((_ROWS, _CHUNK), lambda c: (0, c)),
                pl.BlockSpec((_ROWS, _CHUNK), lambda c: (0, c)),
            ],
            out_specs=[_small_spec(), _small_spec()],
            out_shape=[
                jax.ShapeDtypeStruct((_ROWS, 128), jnp.int32),
                jax.ShapeDtypeStruct((_ROWS, 128), jnp.float32),
            ],
        )(m, s, logits, cst["noise"])
        return bi_f[:, 0]

    return jax.lax.cond(need_fallback, _full, lambda _: idx_fast, None)


# speedup vs baseline: 534.5991x; 1.1319x over previous
import jax
import jax.numpy as jnp
from jax.experimental import pallas as pl

_ROWS = 32
_COLS = 1_000_000
_NOISE_SCALE = 0.1
_CHUNK = 65536
_NCHUNK = -(-_COLS // _CHUNK)
_K = 16
_EPS = 0.02
_IMAX = jnp.iinfo(jnp.int32).max

_consts_cache = None


def _consts():
    global _consts_cache
    if _consts_cache is None:
        with jax.ensure_compile_time_eval():
            nkey = jax.random.fold_in(jax.random.key(0), 1)
            noise = _NOISE_SCALE * jax.random.normal(
                nkey, (_ROWS, _COLS), dtype=jnp.float32)
            tv, ti = jax.lax.top_k(noise, _K + 1)
            delta = tv[:, 0:1] - tv[:, _K:_K + 1]
            lnthr = jnp.log(jnp.maximum(delta, 1e-30)) - _EPS
            _consts_cache = {
                "noise": noise,
                "lnthr": jnp.broadcast_to(lnthr, (_ROWS, 128)),
                "topn": tv[:, :_K],
                "topi": ti[:, :_K].astype(jnp.int32),
            }
    return _consts_cache


def _stream_kernel(x_ref, m_ref, s_ref, cm_ref):
    c = pl.program_id(0)

    @pl.when(c == 0)
    def _():
        m_ref[...] = jnp.full_like(m_ref, -jnp.inf)
        s_ref[...] = jnp.zeros_like(s_ref)

    def _accumulate(x):
        cmax = jnp.max(x, axis=1, keepdims=True)
        m_old = m_ref[...]
        m_new = jnp.maximum(m_old, cmax)
        e = jnp.exp(x - m_new[:, :1])
        csum = jnp.sum(e, axis=1, keepdims=True)
        s_ref[...] = s_ref[...] * jnp.exp(m_old - m_new) + csum
        m_ref[...] = m_new
        cm_ref[0] = jnp.broadcast_to(cmax, cm_ref.shape[1:])

    @pl.when(c < _NCHUNK - 1)
    def _():
        _accumulate(x_ref[...])

    @pl.when(c == _NCHUNK - 1)
    def _():
        x = x_ref[...]
        col = jax.lax.broadcasted_iota(jnp.int32, x.shape, 1) + c * _CHUNK
        _accumulate(jnp.where(col < _COLS, x, -jnp.inf))


def _merge_kernel(m_ref, s_ref, t_ref, cm_ref, xk_ref, nk_ref, ik_ref,
                  out_ref, fb_ref):
    m = m_ref[...][:, :1]
    inv_s = 1.0 / s_ref[...][:, :1]
    theta = m + jnp.log(s_ref[...][:, :1]) + t_ref[...][:, :1]
    lane = jax.lax.broadcasted_iota(jnp.int32, (_ROWS, 128), 1)
    v2 = jnp.where(lane < _K,
                   jnp.exp(xk_ref[...] - m) * inv_s + nk_ref[...],
                   -jnp.inf)
    vm = jnp.max(v2, axis=1, keepdims=True)
    i2 = jnp.min(jnp.where(v2 == vm, ik_ref[...], _IMAX),
                 axis=1, keepdims=True)
    out_ref[...] = jnp.broadcast_to(i2, out_ref.shape)
    fb = jnp.any(cm_ref[...] >= theta)
    fb_ref[...] = jnp.broadcast_to(fb.astype(jnp.int32), fb_ref.shape)


def _argmax_kernel(m_ref, s_ref, x_ref, n_ref, bi_ref, bv_ref):
    c = pl.program_id(0)
    x = x_ref[...]
    n = n_ref[...]
    col = jax.lax.broadcasted_iota(jnp.int32, x.shape, 1) + c * _CHUNK
    m = m_ref[...][:, :1]
    inv_s = 1.0 / s_ref[...][:, :1]
    v = jnp.exp(x - m) * inv_s + n
    v = jnp.where(col < _COLS, v, -jnp.inf)
    cmax = jnp.max(v, axis=1, keepdims=True)
    idxs = jnp.where(v == cmax, col, _IMAX)
    carg = jnp.min(idxs, axis=1, keepdims=True)

    @pl.when(c == 0)
    def _():
        bv_ref[...] = jnp.full_like(bv_ref, -jnp.inf)
        bi_ref[...] = jnp.zeros_like(bi_ref)

    bv = bv_ref[...][:, :1]
    bi = bi_ref[...][:, :1]
    upd = cmax > bv
    bv_ref[...] = jnp.broadcast_to(jnp.where(upd, cmax, bv), bv_ref.shape)
    bi_ref[...] = jnp.broadcast_to(jnp.where(upd, carg, bi), bi_ref.shape)


def _small_spec():
    return pl.BlockSpec((_ROWS, 128), lambda *_: (0, 0))


def kernel(logits):
    cst = _consts()
    m, s, cm = pl.pallas_call(
        _stream_kernel,
        grid=(_NCHUNK,),
        in_specs=[pl.BlockSpec((_ROWS, _CHUNK), lambda c: (0, c))],
        out_specs=[_small_spec(), _small_spec(),
                   pl.BlockSpec((1, _ROWS, 128), lambda c: (c, 0, 0))],
        out_shape=[
            jax.ShapeDtypeStruct((_ROWS, 128), jnp.float32),
            jax.ShapeDtypeStruct((_ROWS, 128), jnp.float32),
            jax.ShapeDtypeStruct((_NCHUNK, _ROWS, 128), jnp.float32),
        ],
    )(logits)

    cm0 = cm[:, :, 0].T
    xk = jnp.take_along_axis(logits, cst["topi"], axis=1)

    def _pad(a, fill, dtype):
        out = jnp.full((_ROWS, 128), fill, dtype)
        return out.at[:, : a.shape[1]].set(a.astype(dtype))

    cm_p = _pad(cm0, -jnp.inf, jnp.float32)
    xk_p = _pad(xk, 0.0, jnp.float32)
    nk_p = _pad(cst["topn"], 0.0, jnp.float32)
    ik_p = _pad(cst["topi"], _IMAX, jnp.int32)

    idx_fast, fb = pl.pallas_call(
        _merge_kernel,
        grid=(1,),
        in_specs=[_small_spec()] * 7,
        out_specs=[_small_spec(), _small_spec()],
        out_shape=[
            jax.ShapeDtypeStruct((_ROWS, 128), jnp.int32),
            jax.ShapeDtypeStruct((_ROWS, 128), jnp.int32),
        ],
    )(m, s, cst["lnthr"], cm_p, xk_p, nk_p, ik_p)
    idx_fast = idx_fast[:, 0]
    need_fallback = fb[0, 0] > 0

    def _full(_):
        bi_f, _bv = pl.pallas_call(
            _argmax_kernel,
            grid=(_NCHUNK,),
            in_specs=[
                _small_spec(), _small_spec(),
                pl.BlockSpec((_ROWS, _CHUNK), lambda c: (0, c)),
                pl.BlockSpec((_ROWS, _CHUNK), lambda c: (0, c)),
            ],
            out_specs=[_small_spec(), _small_spec()],
            out_shape=[
                jax.ShapeDtypeStruct((_ROWS, 128), jnp.int32),
                jax.ShapeDtypeStruct((_ROWS, 128), jnp.float32),
            ],
        )(m, s, logits, cst["noise"])
        return bi_f[:, 0]

    return jax.lax.cond(need_fallback, _full, lambda _: idx_fast, None)


# speedup vs baseline: 548.3770x; 1.0258x over previous
import jax
import jax.numpy as jnp
from jax.experimental import pallas as pl

_ROWS = 32
_COLS = 1_000_000
_NOISE_SCALE = 0.1
_CHUNK = 131072
_NCHUNK = -(-_COLS // _CHUNK)
_CHUNK_FB = 65536
_NCHUNK_FB = -(-_COLS // _CHUNK_FB)
_K = 16
_EPS = 0.02
_IMAX = jnp.iinfo(jnp.int32).max

_consts_cache = None


def _consts():
    global _consts_cache
    if _consts_cache is None:
        with jax.ensure_compile_time_eval():
            nkey = jax.random.fold_in(jax.random.key(0), 1)
            noise = _NOISE_SCALE * jax.random.normal(
                nkey, (_ROWS, _COLS), dtype=jnp.float32)
            tv, ti = jax.lax.top_k(noise, _K + 1)
            delta = tv[:, 0:1] - tv[:, _K:_K + 1]
            lnthr = jnp.log(jnp.maximum(delta, 1e-30)) - _EPS
            _consts_cache = {
                "noise": noise,
                "lnthr": jnp.broadcast_to(lnthr, (_ROWS, 128)),
                "topn": tv[:, :_K],
                "topi": ti[:, :_K].astype(jnp.int32),
            }
    return _consts_cache


def _stream_kernel(x_ref, m_ref, s_ref, cm_ref):
    c = pl.program_id(0)

    @pl.when(c == 0)
    def _():
        m_ref[...] = jnp.full_like(m_ref, -jnp.inf)
        s_ref[...] = jnp.zeros_like(s_ref)

    def _accumulate(x):
        cmax = jnp.max(x, axis=1, keepdims=True)
        m_old = m_ref[...]
        m_new = jnp.maximum(m_old, cmax)
        e = jnp.exp(x - m_new[:, :1])
        csum = jnp.sum(e, axis=1, keepdims=True)
        s_ref[...] = s_ref[...] * jnp.exp(m_old - m_new) + csum
        m_ref[...] = m_new
        cm_ref[0] = jnp.broadcast_to(cmax, cm_ref.shape[1:])

    @pl.when(c < _NCHUNK - 1)
    def _():
        _accumulate(x_ref[...])

    @pl.when(c == _NCHUNK - 1)
    def _():
        x = x_ref[...]
        col = jax.lax.broadcasted_iota(jnp.int32, x.shape, 1) + c * _CHUNK
        _accumulate(jnp.where(col < _COLS, x, -jnp.inf))


def _merge_kernel(m_ref, s_ref, t_ref, cm_ref, xk_ref, nk_ref, ik_ref,
                  out_ref, fb_ref):
    m = m_ref[...][:, :1]
    inv_s = 1.0 / s_ref[...][:, :1]
    theta = m + jnp.log(s_ref[...][:, :1]) + t_ref[...][:, :1]
    lane = jax.lax.broadcasted_iota(jnp.int32, (_ROWS, 128), 1)
    v2 = jnp.where(lane < _K,
                   jnp.exp(xk_ref[...] - m) * inv_s + nk_ref[...],
                   -jnp.inf)
    vm = jnp.max(v2, axis=1, keepdims=True)
    i2 = jnp.min(jnp.where(v2 == vm, ik_ref[...], _IMAX),
                 axis=1, keepdims=True)
    out_ref[...] = jnp.broadcast_to(i2, out_ref.shape)
    fb = jnp.any(cm_ref[...] >= theta)
    fb_ref[...] = jnp.broadcast_to(fb.astype(jnp.int32), fb_ref.shape)


def _argmax_kernel(m_ref, s_ref, x_ref, n_ref, bi_ref, bv_ref):
    c = pl.program_id(0)
    x = x_ref[...]
    n = n_ref[...]
    col = jax.lax.broadcasted_iota(jnp.int32, x.shape, 1) + c * _CHUNK_FB
    m = m_ref[...][:, :1]
    inv_s = 1.0 / s_ref[...][:, :1]
    v = jnp.exp(x - m) * inv_s + n
    v = jnp.where(col < _COLS, v, -jnp.inf)
    cmax = jnp.max(v, axis=1, keepdims=True)
    idxs = jnp.where(v == cmax, col, _IMAX)
    carg = jnp.min(idxs, axis=1, keepdims=True)

    @pl.when(c == 0)
    def _():
        bv_ref[...] = jnp.full_like(bv_ref, -jnp.inf)
        bi_ref[...] = jnp.zeros_like(bi_ref)

    bv = bv_ref[...][:, :1]
    bi = bi_ref[...][:, :1]
    upd = cmax > bv
    bv_ref[...] = jnp.broadcast_to(jnp.where(upd, cmax, bv), bv_ref.shape)
    bi_ref[...] = jnp.broadcast_to(jnp.where(upd, carg, bi), bi_ref.shape)


def _small_spec():
    return pl.BlockSpec((_ROWS, 128), lambda *_: (0, 0))


def kernel(logits):
    cst = _consts()
    m, s, cm = pl.pallas_call(
        _stream_kernel,
        grid=(_NCHUNK,),
        in_specs=[pl.BlockSpec((_ROWS, _CHUNK), lambda c: (0, c))],
        out_specs=[_small_spec(), _small_spec(),
                   pl.BlockSpec((1, _ROWS, 128), lambda c: (c, 0, 0))],
        out_shape=[
            jax.ShapeDtypeStruct((_ROWS, 128), jnp.float32),
            jax.ShapeDtypeStruct((_ROWS, 128), jnp.float32),
            jax.ShapeDtypeStruct((_NCHUNK, _ROWS, 128), jnp.float32),
        ],
    )(logits)

    cm0 = cm[:, :, 0].T
    xk = jnp.take_along_axis(logits, cst["topi"], axis=1)

    def _pad(a, fill, dtype):
        out = jnp.full((_ROWS, 128), fill, dtype)
        return out.at[:, : a.shape[1]].set(a.astype(dtype))

    cm_p = _pad(cm0, -jnp.inf, jnp.float32)
    xk_p = _pad(xk, 0.0, jnp.float32)
    nk_p = _pad(cst["topn"], 0.0, jnp.float32)
    ik_p = _pad(cst["topi"], _IMAX, jnp.int32)

    idx_fast, fb = pl.pallas_call(
        _merge_kernel,
        grid=(1,),
        in_specs=[_small_spec()] * 7,
        out_specs=[_small_spec(), _small_spec()],
        out_shape=[
            jax.ShapeDtypeStruct((_ROWS, 128), jnp.int32),
            jax.ShapeDtypeStruct((_ROWS, 128), jnp.int32),
        ],
    )(m, s, cst["lnthr"], cm_p, xk_p, nk_p, ik_p)
    idx_fast = idx_fast[:, 0]
    need_fallback = fb[0, 0] > 0

    def _full(_):
        bi_f, _bv = pl.pallas_call(
            _argmax_kernel,
            grid=(_NCHUNK_FB,),
            in_specs=[
                _small_spec(), _small_spec(),
                pl.BlockSpec((_ROWS, _CHUNK_FB), lambda c: (0, c)),
                pl.BlockSpec((_ROWS, _CHUNK_FB), lambda c: (0, c)),
            ],
            out_specs=[_small_spec(), _small_spec()],
            out_shape=[
                jax.ShapeDtypeStruct((_ROWS, 128), jnp.int32),
                jax.ShapeDtypeStruct((_ROWS, 128), jnp.float32),
            ],
        )(m, s, logits, cst["noise"])
        return bi_f[:, 0]

    return jax.lax.cond(need_fallback, _full, lambda _: idx_fast, None)
